# baseline (speedup 1.0000x reference)
import functools

import jax
import jax.numpy as jnp
from jax import lax
from jax.experimental import pallas as pl
from jax.experimental.pallas import tpu as pltpu
from jax.experimental.pallas import tpu_sc as plsc

B = 16384
NUM = 13
CAT = 26
V = 100000
D = 64

NC = 2
NS = 16
NW = NC * NS

ROWS_TOTAL = B * CAT
ROWS_PER_W = ROWS_TOTAL // NW
CHUNK = 128
CHUNKS_PER_W = ROWS_PER_W // CHUNK
PERIOD = 13
NBUF = 4


def _sc_gather_body(cat_hbm, tab_hbm, out_hbm, idx_v, offs_v, rows_bufs, sems):
    c = lax.axis_index("c")
    s = lax.axis_index("s")
    wid = s * NC + c

    chunk0 = wid * CHUNKS_PER_W
    pltpu.sync_copy(cat_hbm.at[pl.ds(chunk0 * 1, CHUNKS_PER_W)], idx_v)

    for cc in range(PERIOD):
        for l in range(8):
            p = lax.broadcasted_iota(jnp.int32, (16,), 0) + (cc * CHUNK + l * 16)
            for sub in (832, 416, 208, 104, 52, 26):
                p = jnp.where(p >= sub, p - sub, p)
            offs_v[cc, pl.ds(l * 16, 16)] = p * V

    def fix(k, m):
        for l in range(8):
            sl = pl.ds(l * 16, 16)
            idx_v[k, sl] = idx_v[k, sl] + offs_v[m, sl]
        return jnp.where(m == PERIOD - 1, 0, m + 1)

    lax.fori_loop(0, CHUNKS_PER_W, fix, 0)

    row0 = wid * ROWS_PER_W

    for b in range(NBUF):
        pltpu.async_copy(tab_hbm.at[idx_v.at[b]], rows_bufs[b], sems[b])

    def group(g, _):
        for b in range(NBUF):
            k = g * NBUF + b
            pltpu.make_async_copy(tab_hbm.at[idx_v.at[k]], rows_bufs[b], sems[b]).wait()
            pltpu.sync_copy(rows_bufs[b], out_hbm.at[pl.ds(row0 + k * CHUNK, CHUNK)])

            @pl.when(k + NBUF < CHUNKS_PER_W)
            def _():
                pltpu.async_copy(tab_hbm.at[idx_v.at[k + NBUF]], rows_bufs[b], sems[b])

        return 0

    lax.fori_loop(0, CHUNKS_PER_W // NBUF, group, 0)


def _sc_body_flat(cat_hbm, tab_hbm, out_hbm, idx_v, offs_v, r0, r1, r2, r3,
                  s0, s1, s2, s3):
    _sc_gather_body(cat_hbm, tab_hbm, out_hbm, idx_v, offs_v,
                    [r0, r1, r2, r3], [s0, s1, s2, s3])


@jax.jit
def _sc_gather(cat2d, tabs):
    mesh = plsc.VectorSubcoreMesh(core_axis_name="c", subcore_axis_name="s")
    return pl.kernel(
        _sc_body_flat,
        out_type=jax.ShapeDtypeStruct((ROWS_TOTAL, D), jnp.float32),
        mesh=mesh,
        scratch_types=[
            pltpu.VMEM((CHUNKS_PER_W, CHUNK), jnp.int32),
            pltpu.VMEM((PERIOD, CHUNK), jnp.int32),
        ] + [pltpu.VMEM((CHUNK, D), jnp.float32)] * NBUF
          + [pltpu.SemaphoreType.DMA] * NBUF,
        compiler_params=pltpu.CompilerParams(use_tc_tiling_on_sc=False),
    )(cat2d, tabs)


def _tc_body(num_ref, cat_ref, wn_ref, bn_ref, nw_ref, cw_ref, fp_ref,
             g_ref, be_ref, out_ref):
    x = num_ref[...]
    wn = wn_ref[...] * nw_ref[...]
    bn = bn_ref[...] * nw_ref[...]
    ntok = x[:, :, None] * wn[None, :, :] + bn[None, :, :]
    ctok = cat_ref[...] * cw_ref[...][None, :, :]
    tok = jnp.concatenate([ntok, ctok], axis=1) + fp_ref[...][None, :, :]
    mean = jnp.mean(tok, axis=-1, keepdims=True)
    cen = tok - mean
    var = jnp.mean(cen * cen, axis=-1, keepdims=True)
    y = cen * lax.rsqrt(var + 1e-5)
    out_ref[...] = y * g_ref[...][None, None, :] + be_ref[...][None, None, :]


TB = 256


@jax.jit
def _tc_fuse(num_x, staged, W_num, b_num, num_w, cat_w, feat_pos, gamma, beta):
    grid = (B // TB,)
    return pl.pallas_call(
        _tc_body,
        grid=grid,
        in_specs=[
            pl.BlockSpec((TB, NUM), lambda i: (i, 0)),
            pl.BlockSpec((TB, CAT, D), lambda i: (i, 0, 0)),
            pl.BlockSpec((NUM, D), lambda i: (0, 0)),
            pl.BlockSpec((NUM, D), lambda i: (0, 0)),
            pl.BlockSpec((NUM, 1), lambda i: (0, 0)),
            pl.BlockSpec((CAT, 1), lambda i: (0, 0)),
            pl.BlockSpec((NUM + CAT, D), lambda i: (0, 0)),
            pl.BlockSpec((D,), lambda i: (0,)),
            pl.BlockSpec((D,), lambda i: (0,)),
        ],
        out_specs=pl.BlockSpec((TB, NUM + CAT, D), lambda i: (i, 0, 0)),
        out_shape=jax.ShapeDtypeStruct((B, NUM + CAT, D), jnp.float32),
    )(num_x, staged, W_num, b_num, num_w, cat_w, feat_pos, gamma, beta)


def kernel(num_x, cat_x, W_num, b_num, num_w, cat_tables, cat_w, feat_pos,
           gamma, beta):
    cat2d = cat_x.reshape(ROWS_TOTAL // CHUNK, CHUNK)
    tabs = cat_tables.reshape(CAT * V, D)
    staged = _sc_gather(cat2d, tabs).reshape(B, CAT, D)
    return _tc_fuse(num_x, staged, W_num, b_num, num_w, cat_w, feat_pos,
                    gamma, beta)


# speedup vs baseline: 2.4912x; 2.4912x over previous
import jax
import jax.numpy as jnp
from jax import lax
from jax.experimental import pallas as pl
from jax.experimental.pallas import tpu as pltpu
from jax.experimental.pallas import tpu_sc as plsc

B = 16384
NUM = 13
CAT = 26
V = 100000
D = 64

NC = 2
NS = 16
NW = NC * NS

R_TOTAL = CAT * D
R_PER_W = R_TOTAL // NW
OCHUNK = 4096
NOC = B // OCHUNK


def _sc_body(tab_hbm, catx_hbm, out_hbm, idx_v, row_v, ob0, ob1, sem0, sem1):
    c = lax.axis_index("c")
    s = lax.axis_index("s")
    wid = s * NC + c
    r0 = wid * R_PER_W

    obufs = (ob0, ob1)
    sems = (sem0, sem1)

    def do_row(i, j_prev):
        r = r0 + i
        j = lax.shift_right_logical(r, 6)

        @pl.when(j != j_prev)
        def _():
            pltpu.sync_copy(catx_hbm.at[j], idx_v)

        pltpu.sync_copy(tab_hbm.at[r], row_v)

        def do_chunk(cchunk, _):
            slot = lax.rem(cchunk, 2)

            def per_buf(ob, sm, active):
                @pl.when(active)
                def _():
                    @pl.when(cchunk >= 2)
                    def _():
                        pltpu.make_async_copy(
                            ob, out_hbm.at[r, pl.ds((cchunk - 2) * OCHUNK, OCHUNK)], sm
                        ).wait()

                    def gather16(g, _):
                        base = cchunk * OCHUNK + g * 16
                        idx16 = idx_v[pl.ds(base, 16)]
                        ob[pl.ds(g * 16, 16)] = plsc.load_gather(row_v, [idx16])
                        return 0

                    lax.fori_loop(0, OCHUNK // 16, gather16, 0)
                    pltpu.async_copy(
                        ob, out_hbm.at[r, pl.ds(cchunk * OCHUNK, OCHUNK)], sm
                    )

            per_buf(ob0, sem0, slot == 0)
            per_buf(ob1, sem1, slot == 1)
            return 0

        lax.fori_loop(0, NOC, do_chunk, 0)
        pltpu.make_async_copy(
            ob0, out_hbm.at[r, pl.ds((NOC - 2) * OCHUNK, OCHUNK)], sem0
        ).wait()
        pltpu.make_async_copy(
            ob1, out_hbm.at[r, pl.ds((NOC - 1) * OCHUNK, OCHUNK)], sem1
        ).wait()
        return j

    lax.fori_loop(0, R_PER_W, do_row, jnp.int32(-1))


@jax.jit
def _sc_gather(tab_t, cat_x_t):
    mesh = plsc.VectorSubcoreMesh(core_axis_name="c", subcore_axis_name="s")
    return pl.kernel(
        _sc_body,
        out_type=jax.ShapeDtypeStruct((R_TOTAL, B), jnp.float32),
        mesh=mesh,
        scratch_types=[
            pltpu.VMEM((B,), jnp.int32),
            pltpu.VMEM((V,), jnp.float32),
            pltpu.VMEM((OCHUNK,), jnp.float32),
            pltpu.VMEM((OCHUNK,), jnp.float32),
            pltpu.SemaphoreType.DMA,
            pltpu.SemaphoreType.DMA,
        ],
        compiler_params=pltpu.CompilerParams(use_tc_tiling_on_sc=True,
                                             needs_layout_passes=False),
    )(tab_t, cat_x_t)


def _tc_body(numx_ref, staged_ref, wn_ref, bn_ref, nw_ref, cw_ref, fp_ref,
             g_ref, be_ref, out_ref):
    x = numx_ref[...]
    wn = wn_ref[...] * nw_ref[...]
    bn = bn_ref[...] * nw_ref[...]
    ntok = wn[:, :, None] * x[:, None, :] + bn[:, :, None]
    ctok = staged_ref[...].reshape(CAT, D, -1) * cw_ref[...][:, :, None]
    tok = jnp.concatenate([ntok, ctok], axis=0) + fp_ref[...][:, :, None]
    mean = jnp.mean(tok, axis=1, keepdims=True)
    cen = tok - mean
    var = jnp.mean(cen * cen, axis=1, keepdims=True)
    y = cen * lax.rsqrt(var + 1e-5)
    out_ref[...] = (y * g_ref[...][None, :, None]
                    + be_ref[...][None, :, None])


TB = 512


@jax.jit
def _tc_fuse(num_x_t, staged_t, W_num, b_num, num_w, cat_w, feat_pos, gamma,
             beta):
    grid = (B // TB,)
    out_t = pl.pallas_call(
        _tc_body,
        grid=grid,
        in_specs=[
            pl.BlockSpec((NUM, TB), lambda i: (0, i)),
            pl.BlockSpec((R_TOTAL, TB), lambda i: (0, i)),
            pl.BlockSpec((NUM, D), lambda i: (0, 0)),
            pl.BlockSpec((NUM, D), lambda i: (0, 0)),
            pl.BlockSpec((NUM, 1), lambda i: (0, 0)),
            pl.BlockSpec((CAT, 1), lambda i: (0, 0)),
            pl.BlockSpec((NUM + CAT, D), lambda i: (0, 0)),
            pl.BlockSpec((D,), lambda i: (0,)),
            pl.BlockSpec((D,), lambda i: (0,)),
        ],
        out_specs=pl.BlockSpec((NUM + CAT, D, TB), lambda i: (0, 0, i)),
        out_shape=jax.ShapeDtypeStruct((NUM + CAT, D, B), jnp.float32),
    )(num_x_t, staged_t, W_num, b_num, num_w, cat_w, feat_pos, gamma, beta)
    return jnp.transpose(out_t, (2, 0, 1))


def kernel(num_x, cat_x, W_num, b_num, num_w, cat_tables, cat_w, feat_pos,
           gamma, beta):
    tab_t = jnp.transpose(cat_tables, (0, 2, 1)).reshape(R_TOTAL, V)
    cat_x_t = jnp.transpose(cat_x, (1, 0))
    num_x_t = jnp.transpose(num_x, (1, 0))
    staged_t = _sc_gather(tab_t, cat_x_t)
    return _tc_fuse(num_x_t, staged_t, W_num, b_num, num_w, cat_w, feat_pos,
                    gamma, beta)


# speedup vs baseline: 2.5994x; 1.0434x over previous
import jax
import jax.numpy as jnp
from jax import lax
from jax.experimental import pallas as pl
from jax.experimental.pallas import tpu as pltpu
from jax.experimental.pallas import tpu_sc as plsc

B = 16384
NUM = 13
CAT = 26
V = 100000
D = 64

NC = 2
NS = 16
NW = NC * NS

R_TOTAL = CAT * D
R_PER_W = R_TOTAL // NW
OCHUNK = 4096
NOC = B // OCHUNK


def _sc_body(tab_hbm, catx_hbm, out_hbm, idx_v, row_v, ob0, ob1, sem0, sem1):
    c = lax.axis_index("c")
    s = lax.axis_index("s")
    wid = s * NC + c
    r0 = wid * R_PER_W

    obufs = (ob0, ob1)
    sems = (sem0, sem1)

    def do_row(i, j_prev):
        r = r0 + i
        j = lax.shift_right_logical(r, 6)

        @pl.when(j != j_prev)
        def _():
            pltpu.sync_copy(catx_hbm.at[j], idx_v)

        pltpu.sync_copy(tab_hbm.at[r], row_v)

        def do_chunk(cchunk, _):
            slot = lax.rem(cchunk, 2)

            def per_buf(ob, sm, active):
                @pl.when(active)
                def _():
                    @pl.when(cchunk >= 2)
                    def _():
                        pltpu.make_async_copy(
                            ob, out_hbm.at[r, pl.ds((cchunk - 2) * OCHUNK, OCHUNK)], sm
                        ).wait()

                    def gather128(g, _):
                        for u in range(8):
                            base = g * 128 + u * 16
                            idx16 = idx_v[pl.ds(cchunk * OCHUNK + base, 16)]
                            ob[pl.ds(base, 16)] = plsc.load_gather(row_v, [idx16])
                        return 0

                    lax.fori_loop(0, OCHUNK // 128, gather128, 0)
                    pltpu.async_copy(
                        ob, out_hbm.at[r, pl.ds(cchunk * OCHUNK, OCHUNK)], sm
                    )

            per_buf(ob0, sem0, slot == 0)
            per_buf(ob1, sem1, slot == 1)
            return 0

        lax.fori_loop(0, NOC, do_chunk, 0)
        pltpu.make_async_copy(
            ob0, out_hbm.at[r, pl.ds((NOC - 2) * OCHUNK, OCHUNK)], sem0
        ).wait()
        pltpu.make_async_copy(
            ob1, out_hbm.at[r, pl.ds((NOC - 1) * OCHUNK, OCHUNK)], sem1
        ).wait()
        return j

    lax.fori_loop(0, R_PER_W, do_row, jnp.int32(-1))


@jax.jit
def _sc_gather(tab_t, cat_x_t):
    mesh = plsc.VectorSubcoreMesh(core_axis_name="c", subcore_axis_name="s")
    return pl.kernel(
        _sc_body,
        out_type=jax.ShapeDtypeStruct((R_TOTAL, B), jnp.float32),
        mesh=mesh,
        scratch_types=[
            pltpu.VMEM((B,), jnp.int32),
            pltpu.VMEM((V,), jnp.float32),
            pltpu.VMEM((OCHUNK,), jnp.float32),
            pltpu.VMEM((OCHUNK,), jnp.float32),
            pltpu.SemaphoreType.DMA,
            pltpu.SemaphoreType.DMA,
        ],
        compiler_params=pltpu.CompilerParams(use_tc_tiling_on_sc=True,
                                             needs_layout_passes=False),
    )(tab_t, cat_x_t)


def _tc_body(numx_ref, staged_ref, wn_ref, bn_ref, nw_ref, cw_ref, fp_ref,
             g_ref, be_ref, out_ref):
    x = numx_ref[...]
    wn = wn_ref[...] * nw_ref[...]
    bn = bn_ref[...] * nw_ref[...]
    ntok = wn[:, :, None] * x[:, None, :] + bn[:, :, None]
    ctok = staged_ref[...].reshape(CAT, D, -1) * cw_ref[...][:, :, None]
    tok = jnp.concatenate([ntok, ctok], axis=0) + fp_ref[...][:, :, None]
    mean = jnp.mean(tok, axis=1, keepdims=True)
    cen = tok - mean
    var = jnp.mean(cen * cen, axis=1, keepdims=True)
    y = cen * lax.rsqrt(var + 1e-5)
    out_ref[...] = (y * g_ref[...][None, :, None]
                    + be_ref[...][None, :, None])


TB = 512


@jax.jit
def _tc_fuse(num_x_t, staged_t, W_num, b_num, num_w, cat_w, feat_pos, gamma,
             beta):
    grid = (B // TB,)
    out_t = pl.pallas_call(
        _tc_body,
        grid=grid,
        in_specs=[
            pl.BlockSpec((NUM, TB), lambda i: (0, i)),
            pl.BlockSpec((R_TOTAL, TB), lambda i: (0, i)),
            pl.BlockSpec((NUM, D), lambda i: (0, 0)),
            pl.BlockSpec((NUM, D), lambda i: (0, 0)),
            pl.BlockSpec((NUM, 1), lambda i: (0, 0)),
            pl.BlockSpec((CAT, 1), lambda i: (0, 0)),
            pl.BlockSpec((NUM + CAT, D), lambda i: (0, 0)),
            pl.BlockSpec((D,), lambda i: (0,)),
            pl.BlockSpec((D,), lambda i: (0,)),
        ],
        out_specs=pl.BlockSpec((NUM + CAT, D, TB), lambda i: (0, 0, i)),
        out_shape=jax.ShapeDtypeStruct((NUM + CAT, D, B), jnp.float32),
    )(num_x_t, staged_t, W_num, b_num, num_w, cat_w, feat_pos, gamma, beta)
    return jnp.transpose(out_t, (2, 0, 1))


def kernel(num_x, cat_x, W_num, b_num, num_w, cat_tables, cat_w, feat_pos,
           gamma, beta):
    tab_t = jnp.transpose(cat_tables, (0, 2, 1)).reshape(R_TOTAL, V)
    cat_x_t = jnp.transpose(cat_x, (1, 0))
    num_x_t = jnp.transpose(num_x, (1, 0))
    staged_t = _sc_gather(tab_t, cat_x_t)
    return _tc_fuse(num_x_t, staged_t, W_num, b_num, num_w, cat_w, feat_pos,
                    gamma, beta)
